# baseline (speedup 1.0000x reference)
import functools

import jax
import jax.numpy as jnp
from jax import lax
from jax.experimental import pallas as pl
from jax.experimental.pallas import tpu as pltpu
from jax.experimental.pallas import tpu_sc as plsc

D = 64
BATCH = 4096
HIST = 200
HIST_P = 208
NC = 2
NS = 16
NW = NC * NS
ROWS_PER_W = BATCH // NW
N_OUTER = ROWS_PER_W // 2

_mesh = plsc.VectorSubcoreMesh(core_axis_name="c", subcore_axis_name="s")


@functools.partial(
    pl.kernel,
    mesh=_mesh,
    out_type=jax.ShapeDtypeStruct((BATCH, HIST_P, D), jnp.float32),
    scratch_types=[
        pltpu.VMEM((ROWS_PER_W, HIST), jnp.int32),
        pltpu.VMEM((HIST, D), jnp.float32),
        pltpu.VMEM((HIST, D), jnp.float32),
        pltpu.SemaphoreType.DMA,
        pltpu.SemaphoreType.DMA,
        pltpu.SemaphoreType.DMA,
        pltpu.SemaphoreType.DMA,
    ],
    compiler_params=pltpu.CompilerParams(use_tc_tiling_on_sc=False),
)
def _gather_kernel(idx_hbm, table_hbm, out_hbm,
                   idx_all, rows0, rows1, sg0, sg1, ss0, ss1):
    wid = lax.axis_index("s") * NC + lax.axis_index("c")
    base_w = wid * ROWS_PER_W

    pltpu.sync_copy(idx_hbm.at[pl.ds(base_w, ROWS_PER_W)], idx_all)

    def start_gather(j, rows, sem):
        pltpu.async_copy(table_hbm.at[idx_all.at[j]], rows, sem)

    def wait_gather(rows, sem):
        pltpu.make_async_copy(table_hbm.at[idx_all.at[0]], rows, sem).wait()

    def start_store(j, rows, sem):
        pltpu.async_copy(rows, out_hbm.at[base_w + j, pl.ds(0, HIST)], sem)

    def wait_store(rows, sem):
        pltpu.make_async_copy(rows, out_hbm.at[0, pl.ds(0, HIST)], sem).wait()

    start_gather(0, rows0, sg0)
    start_gather(1, rows1, sg1)
    wait_gather(rows0, sg0)
    start_store(0, rows0, ss0)
    wait_store(rows0, ss0)
    start_gather(2, rows0, sg0)
    wait_gather(rows1, sg1)
    start_store(1, rows1, ss1)

    def body(i, carry):
        j = 2 * i
        wait_store(rows1, ss1)
        start_gather(j + 1, rows1, sg1)
        wait_gather(rows0, sg0)
        start_store(j, rows0, ss0)
        wait_store(rows0, ss0)
        start_gather(j + 2, rows0, sg0)
        wait_gather(rows1, sg1)
        start_store(j + 1, rows1, ss1)
        return carry

    lax.fori_loop(1, N_OUTER - 1, body, 0)

    j = ROWS_PER_W - 2
    wait_store(rows1, ss1)
    start_gather(j + 1, rows1, sg1)
    wait_gather(rows0, sg0)
    start_store(j, rows0, ss0)
    wait_gather(rows1, sg1)
    start_store(j + 1, rows1, ss1)
    wait_store(rows0, ss0)
    wait_store(rows1, ss1)


def kernel(inputs, word_embedding_matrix):
    idx = inputs.astype(jnp.int32)
    out_p = _gather_kernel(idx, word_embedding_matrix)
    return out_p[:, :HIST, :]
